# Initial kernel scaffold; baseline (speedup 1.0000x reference)
import jax
import jax.numpy as jnp
from jax.experimental import pallas as pl


def kernel(inputs, W):
    raise NotImplementedError("write your pallas kernel here")


# speedup vs baseline: 3.1432x; 3.1432x over previous
import functools

import jax
import jax.numpy as jnp
from jax import lax
from jax.experimental import pallas as pl
from jax.experimental.pallas import tpu as pltpu
from jax.experimental.pallas import tpu_sc as plsc

_COMMIT = 0.25
_ROWS_PER_BLOCK = 512
_SC_CHUNK = 128


def _vq_tc_body(nblocks, kdim, x_ref, w_ref, idx_ref, loss_ref):
    f = pl.program_id(0)
    nb = pl.program_id(1)
    x = x_ref[0]
    w = w_ref[0]
    dots = jnp.dot(x, w, preferred_element_type=jnp.float32)
    wsq = jnp.sum(w * w, axis=0, keepdims=True)
    xsq = jnp.sum(x * x, axis=1, keepdims=True)
    d = (xsq - 2.0 * dots) + wsq
    mind = jnp.min(d, axis=1)
    kiota = lax.broadcasted_iota(jnp.int32, d.shape, 1)
    idx = jnp.min(jnp.where(d == mind[:, None], kiota, jnp.int32(kdim)),
                  axis=1)
    idx_ref[0, 0] = idx + f * kdim
    partial = jnp.sum(mind)

    @pl.when(jnp.logical_and(f == 0, nb == 0))
    def _():
        loss_ref[0, 0] = 0.0

    loss_ref[0, 0] += partial


def _vq_assign(inputs, W):
    F, N, D = inputs.shape
    K = W.shape[2]
    Nb = _ROWS_PER_BLOCK
    NB = N // Nb
    idx_out, loss_out = pl.pallas_call(
        functools.partial(_vq_tc_body, NB, K),
        grid=(F, NB),
        in_specs=[
            pl.BlockSpec((1, Nb, D), lambda f, nb: (f, nb, 0)),
            pl.BlockSpec((1, D, K), lambda f, nb: (f, 0, 0)),
        ],
        out_specs=[
            pl.BlockSpec((1, 1, Nb), lambda f, nb: (f * NB + nb, 0, 0)),
            pl.BlockSpec((1, 1), lambda f, nb: (0, 0),
                         memory_space=pltpu.SMEM),
        ],
        out_shape=[
            jax.ShapeDtypeStruct((F * NB, 1, Nb), jnp.int32),
            jax.ShapeDtypeStruct((1, 1), jnp.float32),
        ],
    )(inputs, W)
    return idx_out.reshape(F * N), loss_out[0, 0]


def _sc_gather(table, idx):
    B = idx.shape[0]
    Dd = table.shape[1]
    info = plsc.get_sparse_core_info()
    nc, ns = info.num_cores, info.num_subcores
    nw = nc * ns
    b_per_w = B // nw
    cb = _SC_CHUNK
    n_chunks = b_per_w // cb
    mesh = plsc.VectorSubcoreMesh(core_axis_name="c", subcore_axis_name="s")

    @functools.partial(
        pl.kernel,
        mesh=mesh,
        out_type=jax.ShapeDtypeStruct((B, Dd), jnp.float32),
        scratch_types=[
            pltpu.VMEM((cb,), jnp.int32),
            pltpu.VMEM((cb, Dd), jnp.float32),
            pltpu.SemaphoreType.DMA,
        ],
    )
    def gather_k(table_hbm, idx_hbm, out_hbm, idx_v, rows_v, sem):
        wid = lax.axis_index("s") * nc + lax.axis_index("c")
        base = wid * b_per_w
        for i in range(n_chunks):
            off = base + i * cb
            pltpu.sync_copy(idx_hbm.at[pl.ds(off, cb)], idx_v)
            pltpu.async_copy(table_hbm.at[idx_v], rows_v, sem).wait()
            pltpu.sync_copy(rows_v, out_hbm.at[pl.ds(off, cb)])

    return gather_k(table, idx)


def kernel(inputs, W):
    F, N, D = inputs.shape
    K = W.shape[2]
    idx_flat, loss_sum = _vq_assign(inputs, W)
    wt = jnp.swapaxes(W, 1, 2).reshape(F * K, D)
    quantized = _sc_gather(wt, idx_flat).reshape(F, N, D)
    loss = loss_sum * ((1.0 + _COMMIT) / (F * N * D))
    return quantized, loss


# speedup vs baseline: 3.3399x; 1.0626x over previous
import functools

import jax
import jax.numpy as jnp
from jax import lax
from jax.experimental import pallas as pl
from jax.experimental.pallas import tpu as pltpu
from jax.experimental.pallas import tpu_sc as plsc

_COMMIT = 0.25
_LANES = 128
_ROWS_PER_BLOCK = 1024
_SC_CHUNK = 128


def _vq_tc_body(nblocks, kdim, x_ref, w_ref, idx_ref, loss_ref):
    f = pl.program_id(0)
    nb = pl.program_id(1)
    x = x_ref[0]
    w = w_ref[0]
    dots = jnp.dot(x, w, preferred_element_type=jnp.float32)
    wsq = jnp.sum(w * w, axis=0, keepdims=True)
    xsq = jnp.sum(x * x, axis=1, keepdims=True)
    ngrp = kdim // _LANES
    minval = (xsq - 2.0 * dots[:, 0:_LANES]) + wsq[:, 0:_LANES]
    jwin = jnp.zeros(minval.shape, jnp.int32)
    for j in range(1, ngrp):
        sl = slice(j * _LANES, (j + 1) * _LANES)
        dj = (xsq - 2.0 * dots[:, sl]) + wsq[:, sl]
        better = dj < minval
        minval = jnp.where(better, dj, minval)
        jwin = jnp.where(better, jnp.int32(j), jwin)
    mind = jnp.min(minval, axis=1)
    liota = lax.broadcasted_iota(jnp.int32, minval.shape, 1)
    kcand = jwin * _LANES + liota
    masked = jnp.where(minval == mind[:, None], kcand, jnp.int32(kdim))
    idx = jnp.min(masked, axis=1)
    idx_ref[0, 0] = idx + f * kdim
    partial = jnp.sum(mind)

    @pl.when(jnp.logical_and(f == 0, nb == 0))
    def _():
        loss_ref[0, 0] = 0.0

    loss_ref[0, 0] += partial


def _vq_assign(inputs, W):
    F, N, D = inputs.shape
    K = W.shape[2]
    Nb = _ROWS_PER_BLOCK
    NB = N // Nb
    idx_out, loss_out = pl.pallas_call(
        functools.partial(_vq_tc_body, NB, K),
        grid=(F, NB),
        in_specs=[
            pl.BlockSpec((1, Nb, D), lambda f, nb: (f, nb, 0)),
            pl.BlockSpec((1, D, K), lambda f, nb: (f, 0, 0)),
        ],
        out_specs=[
            pl.BlockSpec((1, 1, Nb), lambda f, nb: (f * NB + nb, 0, 0)),
            pl.BlockSpec((1, 1), lambda f, nb: (0, 0),
                         memory_space=pltpu.SMEM),
        ],
        out_shape=[
            jax.ShapeDtypeStruct((F * NB, 1, Nb), jnp.int32),
            jax.ShapeDtypeStruct((1, 1), jnp.float32),
        ],
    )(inputs, W)
    return idx_out.reshape(F * N), loss_out[0, 0]


def _sc_gather(table, idx):
    B = idx.shape[0]
    Dd = table.shape[1]
    info = plsc.get_sparse_core_info()
    nc, ns = info.num_cores, info.num_subcores
    nw = nc * ns
    b_per_w = B // nw
    cb = _SC_CHUNK
    n_chunks = b_per_w // cb
    mesh = plsc.VectorSubcoreMesh(core_axis_name="c", subcore_axis_name="s")

    @functools.partial(
        pl.kernel,
        mesh=mesh,
        out_type=jax.ShapeDtypeStruct((B, Dd), jnp.float32),
        scratch_types=[
            pltpu.VMEM((cb,), jnp.int32),
            pltpu.VMEM((cb, Dd), jnp.float32),
            pltpu.SemaphoreType.DMA,
        ],
    )
    def gather_k(table_hbm, idx_hbm, out_hbm, idx_v, rows_v, sem):
        wid = lax.axis_index("s") * nc + lax.axis_index("c")
        base = wid * b_per_w
        for i in range(n_chunks):
            off = base + i * cb
            pltpu.sync_copy(idx_hbm.at[pl.ds(off, cb)], idx_v)
            pltpu.async_copy(table_hbm.at[idx_v], rows_v, sem).wait()
            pltpu.sync_copy(rows_v, out_hbm.at[pl.ds(off, cb)])

    return gather_k(table, idx)


def kernel(inputs, W):
    F, N, D = inputs.shape
    K = W.shape[2]
    idx_flat, loss_sum = _vq_assign(inputs, W)
    wt = jnp.swapaxes(W, 1, 2).reshape(F * K, D)
    quantized = _sc_gather(wt, idx_flat).reshape(F, N, D)
    loss = loss_sum * ((1.0 + _COMMIT) / (F * N * D))
    return quantized, loss
